# baseline (speedup 1.0000x reference)
import jax
import jax.numpy as jnp
from jax import lax
from jax.experimental import pallas as pl
from jax.experimental.pallas import tpu as pltpu

_KN = 8192
_D = 32
_N = 100000
_NITER = 1000
_A_START = 0.05
_A_END = 0.01
_THR = 0.5
_IBIG = 2**31 - 1
_S = 8
_RB = _KN // _S


def _tc_body(ridx_s, k_s, dataT_h, nodesT_h, nhbr_h, out_h,
             xiblk_v, ntbuf_v, colblk_v, obuf_v, sem1, semn, csems, osems):
    k = k_s[0]
    i = ridx_s[k]
    alpha = jnp.float32(_A_START) - jnp.float32(_A_START - _A_END) * (
        k.astype(jnp.float32) / _NITER)

    cpn = pltpu.make_async_copy(nodesT_h, ntbuf_v, semn)
    cpn.start()

    ib = i // 128
    cp1 = pltpu.make_async_copy(
        dataT_h.at[:, pl.ds(ib * 128, 128)], xiblk_v, sem1)
    cp1.start()
    cp1.wait()
    j = i - ib * 128
    lane = lax.broadcasted_iota(jnp.int32, (1, 128), 1)
    xcol = jnp.sum(jnp.where(lane == j, xiblk_v[...], 0.0),
                   axis=1, keepdims=True)

    cpn.wait()
    nt = ntbuf_v[...]
    diff = nt - xcol
    dist2 = jnp.sum(diff * diff, axis=0, keepdims=True)
    m = jnp.min(dist2)
    nio = lax.broadcasted_iota(jnp.int32, (1, _KN), 1)
    nearest = jnp.min(jnp.where(dist2 == m, nio, _IBIG))

    cb = nearest // 128
    cps = []
    for c in range(_S):
        cp = pltpu.make_async_copy(
            nhbr_h.at[pl.ds(c * _RB, _RB), pl.ds(cb * 128, 128)],
            colblk_v.at[pl.ds(c * _RB, _RB)], csems.at[c])
        cp.start()
        cps.append(cp)
    jc = nearest - cb * 128
    onehot2 = (lane == jc).astype(jnp.float32)
    ocps = []
    for c in range(_S):
        if c >= 2:
            ocps[c - 2].wait()
        cps[c].wait()
        mchunk = (colblk_v[pl.ds(c * _RB, _RB), :] <= _THR).astype(jnp.float32)
        colm = lax.dot_general(onehot2, mchunk, (((1,), (1,)), ((), ())),
                               preferred_element_type=jnp.float32)
        am = colm * alpha
        ntc = nt[:, c * _RB:(c + 1) * _RB]
        obuf_v[c % 2] = ntc + (xcol - ntc) * am
        ocp = pltpu.make_async_copy(
            obuf_v.at[c % 2], out_h.at[:, pl.ds(c * _RB, _RB)],
            osems.at[c % 2])
        ocp.start()
        ocps.append(ocp)
    ocps[_S - 2].wait()
    ocps[_S - 1].wait()


_tc_call = pl.pallas_call(
    _tc_body,
    out_shape=jax.ShapeDtypeStruct((_D, _KN), jnp.float32),
    in_specs=[
        pl.BlockSpec(memory_space=pltpu.SMEM),
        pl.BlockSpec(memory_space=pltpu.SMEM),
        pl.BlockSpec(memory_space=pltpu.HBM),
        pl.BlockSpec(memory_space=pltpu.HBM),
        pl.BlockSpec(memory_space=pltpu.HBM),
    ],
    out_specs=pl.BlockSpec(memory_space=pltpu.HBM),
    scratch_shapes=[
        pltpu.VMEM((_D, 128), jnp.float32),
        pltpu.VMEM((_D, _KN), jnp.float32),
        pltpu.VMEM((_KN, 128), jnp.float32),
        pltpu.VMEM((2, _D, _RB), jnp.float32),
        pltpu.SemaphoreType.DMA,
        pltpu.SemaphoreType.DMA,
        pltpu.SemaphoreType.DMA((_S,)),
        pltpu.SemaphoreType.DMA((2,)),
    ],
    compiler_params=pltpu.CompilerParams(
        dimension_semantics=(), vmem_limit_bytes=100 * 1024 * 1024),
)


def kernel(data, nodes, nhbrdist, rand_indices, k):
    karr = jnp.reshape(k, (1,)).astype(jnp.int32)
    out_t = _tc_call(rand_indices, karr, data.T, nodes.T, nhbrdist)
    return out_t.T


# speedup vs baseline: 1.2103x; 1.2103x over previous
import jax
import jax.numpy as jnp
from jax import lax
from jax.experimental import pallas as pl
from jax.experimental.pallas import tpu as pltpu

_KN = 8192
_D = 32
_N = 100000
_NITER = 1000
_A_START = 0.05
_A_END = 0.01
_THR = 0.5
_IBIG = 2**31 - 1
_S = 4
_RB = _KN // _S


def _tc_body(ridx_s, k_s, dataT_h, nodesT_h, nhbr_h, out_v,
             xiblk_v, ntbuf_v, colblk_v, sem1, semn, csems):
    k = k_s[0]
    i = ridx_s[k]
    alpha = jnp.float32(_A_START) - jnp.float32(_A_START - _A_END) * (
        k.astype(jnp.float32) / _NITER)

    cpn = pltpu.make_async_copy(nodesT_h, ntbuf_v, semn)
    cpn.start()

    ib = i // 128
    cp1 = pltpu.make_async_copy(
        dataT_h.at[:, pl.ds(ib * 128, 128)], xiblk_v, sem1)
    cp1.start()
    cp1.wait()
    j = i - ib * 128
    lane = lax.broadcasted_iota(jnp.int32, (1, 128), 1)
    xcol = jnp.sum(jnp.where(lane == j, xiblk_v[...], 0.0),
                   axis=1, keepdims=True)

    cpn.wait()
    nt = ntbuf_v[...]
    diff = nt - xcol
    dist2 = jnp.sum(diff * diff, axis=0, keepdims=True)
    m = jnp.min(dist2)
    nio = lax.broadcasted_iota(jnp.int32, (1, _KN), 1)
    nearest = jnp.min(jnp.where(dist2 == m, nio, _IBIG))

    cb = nearest // 128
    cps = []
    for c in range(_S):
        cp = pltpu.make_async_copy(
            nhbr_h.at[pl.ds(c * _RB, _RB), pl.ds(cb * 128, 128)],
            colblk_v.at[pl.ds(c * _RB, _RB)], csems.at[c])
        cp.start()
        cps.append(cp)
    jc = nearest - cb * 128
    onehot2 = (lane == jc).astype(jnp.float32)
    for c in range(_S):
        cps[c].wait()
        mchunk = (colblk_v[pl.ds(c * _RB, _RB), :] <= _THR).astype(jnp.float32)
        colm = lax.dot_general(onehot2, mchunk, (((1,), (1,)), ((), ())),
                               preferred_element_type=jnp.float32)
        am = colm * alpha
        ntc = nt[:, c * _RB:(c + 1) * _RB]
        out_v[:, pl.ds(c * _RB, _RB)] = ntc + (xcol - ntc) * am


_tc_call = pl.pallas_call(
    _tc_body,
    out_shape=jax.ShapeDtypeStruct((_D, _KN), jnp.float32),
    in_specs=[
        pl.BlockSpec(memory_space=pltpu.SMEM),
        pl.BlockSpec(memory_space=pltpu.SMEM),
        pl.BlockSpec(memory_space=pltpu.HBM),
        pl.BlockSpec(memory_space=pltpu.HBM),
        pl.BlockSpec(memory_space=pltpu.HBM),
    ],
    out_specs=pl.BlockSpec(memory_space=pltpu.VMEM),
    scratch_shapes=[
        pltpu.VMEM((_D, 128), jnp.float32),
        pltpu.VMEM((_D, _KN), jnp.float32),
        pltpu.VMEM((_KN, 128), jnp.float32),
        pltpu.SemaphoreType.DMA,
        pltpu.SemaphoreType.DMA,
        pltpu.SemaphoreType.DMA((_S,)),
    ],
    compiler_params=pltpu.CompilerParams(
        dimension_semantics=(), vmem_limit_bytes=100 * 1024 * 1024),
)


def kernel(data, nodes, nhbrdist, rand_indices, k):
    karr = jnp.reshape(k, (1,)).astype(jnp.int32)
    out_t = _tc_call(rand_indices, karr, data.T, nodes.T, nhbrdist)
    return out_t.T
